# baseline (speedup 1.0000x reference)
import functools

import jax
import jax.numpy as jnp
from jax import lax
from jax.experimental import pallas as pl
from jax.experimental.pallas import tpu as pltpu
from jax.experimental.pallas import tpu_sc as plsc

N_NODES = 10000
N_EDGES = 320000
D_FEAT = 128
MAX_HOP = 3

NC = 2
NS = 16
N_PAD = 10240
NODES_PER_CORE = N_PAD // NC
ROWS_PER_TILE = NODES_PER_CORE // NS
DUMP_ROW = NODES_PER_CORE
ACC_ROWS = NODES_PER_CORE + 8
EDGES_PER_TILE = N_EDGES // NS
CHUNK = 128
CHUNKS_RUN = 158
CHUNKS_PAD = CHUNKS_RUN + 2


def _sc_body(table, src_idx, dst_idx, zeros, out, acc, rows, srci, dsti,
             sem0, sem1):
    c = lax.axis_index("c")
    s = lax.axis_index("s")
    sems = (sem0, sem1)

    pltpu.sync_copy(src_idx.at[c, s], srci)
    pltpu.sync_copy(dst_idx.at[s], dsti)

    row0 = s * ROWS_PER_TILE
    out_row0 = c * NODES_PER_CORE + s * ROWS_PER_TILE

    pltpu.sync_copy(zeros, acc.at[pl.ds(row0, ROWS_PER_TILE)])
    plsc.subcore_barrier()

    def drain(b):
        pltpu.make_async_copy(zeros.at[pl.ds(0, CHUNK)], rows.at[b],
                              sems[b]).wait()

    pltpu.async_copy(table.at[dsti.at[0]], rows.at[0], sems[0])

    def ring_body(i, _):
        for b in range(2):
            k = 2 * i + b
            pltpu.async_copy(table.at[dsti.at[k + 1]], rows.at[1 - b],
                             sems[1 - b])
            drain(b)
            pltpu.sync_copy(rows.at[b], acc.at[srci.at[k]], add=True)
        return ()

    lax.fori_loop(0, CHUNKS_RUN // 2, ring_body, ())
    drain(0)

    plsc.subcore_barrier()
    pltpu.sync_copy(acc.at[pl.ds(row0, ROWS_PER_TILE)],
                    out.at[pl.ds(out_row0, ROWS_PER_TILE)])


@functools.partial(
    pl.kernel,
    out_type=jax.ShapeDtypeStruct((N_PAD, D_FEAT), jnp.float32),
    mesh=plsc.VectorSubcoreMesh(core_axis_name="c", subcore_axis_name="s"),
    scratch_types=[
        pltpu.VMEM_SHARED((ACC_ROWS, D_FEAT), jnp.float32),
        pltpu.VMEM((2, CHUNK, D_FEAT), jnp.float32),
        pltpu.VMEM((CHUNKS_RUN, CHUNK), jnp.int32),
        pltpu.VMEM((CHUNKS_PAD, CHUNK), jnp.int32),
        pltpu.SemaphoreType.DMA,
        pltpu.SemaphoreType.DMA,
    ],
)
def _hop_kernel(table, src_idx, dst_idx, zeros, out, acc, rows, srci, dsti,
                sem0, sem1):
    _sc_body(table, src_idx, dst_idx, zeros, out, acc, rows, srci, dsti,
             sem0, sem1)


@functools.partial(
    pl.kernel,
    out_type=jax.ShapeDtypeStruct((N_PAD, D_FEAT), jnp.float32),
    mesh=plsc.VectorSubcoreMesh(core_axis_name="c", subcore_axis_name="s"),
    scratch_types=[pltpu.VMEM((ROWS_PER_TILE, D_FEAT), jnp.float32)],
)
def _linearize(xp, out, buf):
    c = lax.axis_index("c")
    s = lax.axis_index("s")
    r0 = (s * NC + c) * ROWS_PER_TILE
    pltpu.sync_copy(xp.at[pl.ds(r0, ROWS_PER_TILE)], buf)
    pltpu.sync_copy(buf, out.at[pl.ds(r0, ROWS_PER_TILE)])


def kernel(x, edge_index, num_nodes):
    del num_nodes
    src = edge_index[0]
    dst = edge_index[1]
    core = src // NODES_PER_CORE
    local = src - core * NODES_PER_CORE
    srcm = jnp.stack(
        [jnp.where(core == c, local, DUMP_ROW) for c in range(NC)]
    ).reshape(NC, NS, EDGES_PER_TILE)
    srcm = jnp.pad(srcm,
                   ((0, 0), (0, 0), (0, CHUNKS_RUN * CHUNK - EDGES_PER_TILE)),
                   constant_values=DUMP_ROW)
    srcm = srcm.reshape(NC, NS, CHUNKS_RUN, CHUNK)
    dst3 = dst.reshape(NS, EDGES_PER_TILE)
    dst3 = jnp.pad(dst3, ((0, 0), (0, CHUNKS_PAD * CHUNK - EDGES_PER_TILE)))
    dst3 = dst3.reshape(NS, CHUNKS_PAD, CHUNK)
    zeros = jnp.zeros((ROWS_PER_TILE, D_FEAT), jnp.float32)

    table = _linearize(jnp.pad(x, ((0, N_PAD - N_NODES), (0, 0))))
    hops = []
    for _ in range(MAX_HOP):
        table = _hop_kernel(table, srcm, dst3, zeros)
        hops.append(table[:N_NODES])
    y = jnp.stack(hops)
    return jnp.concatenate([x[:, None], jnp.transpose(y, (1, 0, 2))], axis=1)


# speedup vs baseline: 1.8871x; 1.8871x over previous
import functools

import jax
import jax.numpy as jnp
from jax import lax
from jax.experimental import pallas as pl
from jax.experimental.pallas import tpu as pltpu
from jax.experimental.pallas import tpu_sc as plsc

N_NODES = 10000
N_EDGES = 320000
D_FEAT = 128
MAX_HOP = 3

NC = 2
NS = 16
N_PAD = 10240
NODES_PER_CORE = N_PAD // NC
ROWS_PER_TILE = NODES_PER_CORE // NS
DUMP_ROW = NODES_PER_CORE
ACC_ROWS = NODES_PER_CORE + 8
EDGES_PER_TILE = N_EDGES // NS
CHUNK = 80
CHUNKS_RUN = EDGES_PER_TILE // CHUNK
CHUNKS_PAD = CHUNKS_RUN + 2


def _sc_body(table, src_idx, dst_idx, zeros, out, acc, rows, srci, dsti,
             sem0, sem1):
    c = lax.axis_index("c")
    s = lax.axis_index("s")
    sems = (sem0, sem1)

    pltpu.sync_copy(src_idx.at[c, s], srci)
    pltpu.sync_copy(dst_idx.at[s], dsti)

    row0 = s * ROWS_PER_TILE
    out_row0 = c * NODES_PER_CORE + s * ROWS_PER_TILE

    pltpu.sync_copy(zeros, acc.at[pl.ds(row0, ROWS_PER_TILE)])
    plsc.subcore_barrier()

    def drain(b):
        pltpu.make_async_copy(zeros.at[pl.ds(0, CHUNK)], rows.at[b],
                              sems[b]).wait()

    pltpu.async_copy(table.at[dsti.at[0]], rows.at[0], sems[0])

    def ring_body(i, _):
        for b in range(2):
            k = 2 * i + b
            pltpu.async_copy(table.at[dsti.at[k + 1]], rows.at[1 - b],
                             sems[1 - b])
            drain(b)
            pltpu.sync_copy(rows.at[b], acc.at[srci.at[k]], add=True)
        return ()

    lax.fori_loop(0, CHUNKS_RUN // 2, ring_body, ())
    drain(0)

    plsc.subcore_barrier()
    pltpu.sync_copy(acc.at[pl.ds(row0, ROWS_PER_TILE)],
                    out.at[pl.ds(out_row0, ROWS_PER_TILE)])


@functools.partial(
    pl.kernel,
    out_type=jax.ShapeDtypeStruct((N_PAD, D_FEAT), jnp.float32),
    mesh=plsc.VectorSubcoreMesh(core_axis_name="c", subcore_axis_name="s"),
    scratch_types=[
        pltpu.VMEM_SHARED((ACC_ROWS, D_FEAT), jnp.float32),
        pltpu.VMEM((2, CHUNK, D_FEAT), jnp.float32),
        pltpu.VMEM((CHUNKS_RUN, CHUNK), jnp.int32),
        pltpu.VMEM((CHUNKS_PAD, CHUNK), jnp.int32),
        pltpu.SemaphoreType.DMA,
        pltpu.SemaphoreType.DMA,
    ],
)
def _hop_kernel(table, src_idx, dst_idx, zeros, out, acc, rows, srci, dsti,
                sem0, sem1):
    _sc_body(table, src_idx, dst_idx, zeros, out, acc, rows, srci, dsti,
             sem0, sem1)


@functools.partial(
    pl.kernel,
    out_type=jax.ShapeDtypeStruct((N_PAD, D_FEAT), jnp.float32),
    mesh=plsc.VectorSubcoreMesh(core_axis_name="c", subcore_axis_name="s"),
    scratch_types=[pltpu.VMEM((ROWS_PER_TILE, D_FEAT), jnp.float32)],
)
def _linearize(xp, out, buf):
    c = lax.axis_index("c")
    s = lax.axis_index("s")
    r0 = (s * NC + c) * ROWS_PER_TILE
    pltpu.sync_copy(xp.at[pl.ds(r0, ROWS_PER_TILE)], buf)
    pltpu.sync_copy(buf, out.at[pl.ds(r0, ROWS_PER_TILE)])


def kernel(x, edge_index, num_nodes):
    del num_nodes
    src = edge_index[0]
    dst = edge_index[1]
    core = src // NODES_PER_CORE
    local = src - core * NODES_PER_CORE
    srcm = jnp.stack(
        [jnp.where(core == c, local, DUMP_ROW) for c in range(NC)]
    ).reshape(NC, NS, EDGES_PER_TILE)
    srcm = jnp.pad(srcm,
                   ((0, 0), (0, 0), (0, CHUNKS_RUN * CHUNK - EDGES_PER_TILE)),
                   constant_values=DUMP_ROW)
    srcm = srcm.reshape(NC, NS, CHUNKS_RUN, CHUNK)
    dst3 = dst.reshape(NS, EDGES_PER_TILE)
    dst3 = jnp.pad(dst3, ((0, 0), (0, CHUNKS_PAD * CHUNK - EDGES_PER_TILE)))
    dst3 = dst3.reshape(NS, CHUNKS_PAD, CHUNK)
    zeros = jnp.zeros((ROWS_PER_TILE, D_FEAT), jnp.float32)

    table = _linearize(jnp.pad(x, ((0, N_PAD - N_NODES), (0, 0))))
    hops = []
    for _ in range(MAX_HOP):
        table = _hop_kernel(table, srcm, dst3, zeros)
        hops.append(table[:N_NODES])
    y = jnp.stack(hops)
    return jnp.concatenate([x[:, None], jnp.transpose(y, (1, 0, 2))], axis=1)
